# baseline (speedup 1.0000x reference)
import functools

import jax
import jax.numpy as jnp
from jax.experimental import pallas as pl
from jax.experimental.pallas import tpu as pltpu

_HC = 2048
_BC = 128


def _fused_body(nh, x_ref, w1_ref, b1c_ref, w2_ref, b2_ref, o_ref,
                mct_ref, bct_ref, bias_ref):
    g = pl.program_id(0)

    @pl.when(g < nh)
    def _compose():
        w2c = w2_ref[...]
        part = jax.lax.dot_general(
            w2c, w1_ref[...], dimension_numbers=(((1,), (0,)), ((), ())),
            preferred_element_type=jnp.float32,
        )
        partb = jax.lax.dot_general(
            w2c, b1c_ref[...], dimension_numbers=(((1,), (0,)), ((), ())),
            preferred_element_type=jnp.float32,
        )

        @pl.when(g == 0)
        def _():
            mct_ref[...] = part
            bct_ref[...] = partb

        @pl.when(g > 0)
        def _():
            mct_ref[...] += part
            bct_ref[...] += partb

    @pl.when(g >= nh)
    def _apply():
        @pl.when(g == nh)
        def _():
            bias_ref[...] = jnp.transpose(bct_ref[:, :1]) + b2_ref[...]

        acc = jax.lax.dot_general(
            x_ref[...], mct_ref[...], dimension_numbers=(((1,), (1,)), ((), ())),
            preferred_element_type=jnp.float32,
        )
        o_ref[...] = acc + bias_ref[...]


def kernel(x, w1, b1, w2, b2):
    B, I = x.shape
    H = w1.shape[0]
    O = w2.shape[0]

    nh = H // _HC
    TB = min(512, B)
    nb = B // TB
    b1c = jnp.zeros((H, _BC), jnp.float32).at[:, 0].set(b1)

    out = pl.pallas_call(
        functools.partial(_fused_body, nh),
        out_shape=jax.ShapeDtypeStruct((B, O), jnp.float32),
        grid=(nh + nb,),
        in_specs=[
            pl.BlockSpec((TB, I), lambda g: (jnp.maximum(g - nh, 0), 0)),
            pl.BlockSpec((_HC, I), lambda g: (jnp.minimum(g, nh - 1), 0)),
            pl.BlockSpec((_HC, _BC), lambda g: (jnp.minimum(g, nh - 1), 0)),
            pl.BlockSpec((O, _HC), lambda g: (0, jnp.minimum(g, nh - 1))),
            pl.BlockSpec((1, O), lambda g: (0, 0)),
        ],
        out_specs=pl.BlockSpec((TB, O), lambda g: (jnp.maximum(g - nh, 0), 0)),
        scratch_shapes=[
            pltpu.VMEM((O, I), jnp.float32),
            pltpu.VMEM((O, _BC), jnp.float32),
            pltpu.VMEM((1, O), jnp.float32),
        ],
        compiler_params=pltpu.CompilerParams(
            dimension_semantics=("arbitrary",),
        ),
    )(x, w1, b1c, w2, b2.reshape(1, O).astype(jnp.float32))
    return out


# speedup vs baseline: 1.0286x; 1.0286x over previous
import functools

import jax
import jax.numpy as jnp
from jax.experimental import pallas as pl
from jax.experimental.pallas import tpu as pltpu

_HC = 1024
_BC = 128


def _fused_body(nh, x_ref, w1_ref, b1c_ref, w2_ref, b2_ref, o_ref,
                mct_ref, bct_ref, bias_ref):
    g = pl.program_id(0)

    @pl.when(g < nh)
    def _compose():
        w2c = w2_ref[...]
        part = jax.lax.dot_general(
            w2c, w1_ref[...], dimension_numbers=(((1,), (0,)), ((), ())),
            preferred_element_type=jnp.float32,
        )
        partb = jax.lax.dot_general(
            w2c, b1c_ref[...], dimension_numbers=(((1,), (0,)), ((), ())),
            preferred_element_type=jnp.float32,
        )

        @pl.when(g == 0)
        def _():
            mct_ref[...] = part
            bct_ref[...] = partb

        @pl.when(g > 0)
        def _():
            mct_ref[...] += part
            bct_ref[...] += partb

    @pl.when(g >= nh)
    def _apply():
        @pl.when(g == nh)
        def _():
            bias_ref[...] = jnp.transpose(bct_ref[:, :1]) + b2_ref[...]

        acc = jax.lax.dot_general(
            x_ref[...], mct_ref[...], dimension_numbers=(((1,), (1,)), ((), ())),
            preferred_element_type=jnp.float32,
        )
        o_ref[...] = acc + bias_ref[...]


def kernel(x, w1, b1, w2, b2):
    B, I = x.shape
    H = w1.shape[0]
    O = w2.shape[0]

    nh = H // _HC
    TB = min(512, B)
    nb = B // TB
    b1c = jnp.zeros((H, _BC), jnp.float32).at[:, 0].set(b1)

    out = pl.pallas_call(
        functools.partial(_fused_body, nh),
        out_shape=jax.ShapeDtypeStruct((B, O), jnp.float32),
        grid=(nh + nb,),
        in_specs=[
            pl.BlockSpec((TB, I), lambda g: (jnp.maximum(g - nh, 0), 0)),
            pl.BlockSpec((_HC, I), lambda g: (jnp.minimum(g, nh - 1), 0)),
            pl.BlockSpec((_HC, _BC), lambda g: (jnp.minimum(g, nh - 1), 0)),
            pl.BlockSpec((O, _HC), lambda g: (0, jnp.minimum(g, nh - 1))),
            pl.BlockSpec((1, O), lambda g: (0, 0)),
        ],
        out_specs=pl.BlockSpec((TB, O), lambda g: (jnp.maximum(g - nh, 0), 0)),
        scratch_shapes=[
            pltpu.VMEM((O, I), jnp.float32),
            pltpu.VMEM((O, _BC), jnp.float32),
            pltpu.VMEM((1, O), jnp.float32),
        ],
        compiler_params=pltpu.CompilerParams(
            dimension_semantics=("arbitrary",),
        ),
    )(x, w1, b1c, w2, b2.reshape(1, O).astype(jnp.float32))
    return out


# speedup vs baseline: 1.0615x; 1.0321x over previous
import functools

import jax
import jax.numpy as jnp
from jax.experimental import pallas as pl
from jax.experimental.pallas import tpu as pltpu

_HC = 1024
_BC = 128


def _fused_body(nh, x_ref, w1_ref, b1c_ref, w2_ref, b2_ref, o_ref,
                mct_ref, bct_ref, mcb_ref, bias_ref):
    g = pl.program_id(0)

    @pl.when(g < nh)
    def _compose():
        w2c = w2_ref[...].astype(jnp.bfloat16)
        w1c = w1_ref[...].astype(jnp.bfloat16)
        part = jax.lax.dot_general(
            w2c, w1c, dimension_numbers=(((1,), (0,)), ((), ())),
            preferred_element_type=jnp.float32,
        )
        partb = jax.lax.dot_general(
            w2c, b1c_ref[...], dimension_numbers=(((1,), (0,)), ((), ())),
            preferred_element_type=jnp.float32,
        )

        @pl.when(g == 0)
        def _():
            mct_ref[...] = part
            bct_ref[...] = partb

        @pl.when(g > 0)
        def _():
            mct_ref[...] += part
            bct_ref[...] += partb

    @pl.when(g >= nh)
    def _apply():
        @pl.when(g == nh)
        def _():
            mcb_ref[...] = mct_ref[...].astype(jnp.bfloat16)
            bias_ref[...] = jnp.transpose(bct_ref[:, :1]) + b2_ref[...]

        xb = x_ref[...].astype(jnp.bfloat16)
        acc = jax.lax.dot_general(
            xb, mcb_ref[...], dimension_numbers=(((1,), (1,)), ((), ())),
            preferred_element_type=jnp.float32,
        )
        o_ref[...] = acc + bias_ref[...]


def kernel(x, w1, b1, w2, b2):
    B, I = x.shape
    H = w1.shape[0]
    O = w2.shape[0]

    nh = H // _HC
    TB = min(1024, B)
    nb = B // TB
    b1c = jnp.zeros((H, _BC), jnp.bfloat16).at[:, 0].set(b1.astype(jnp.bfloat16))

    out = pl.pallas_call(
        functools.partial(_fused_body, nh),
        out_shape=jax.ShapeDtypeStruct((B, O), jnp.float32),
        grid=(nh + nb,),
        in_specs=[
            pl.BlockSpec((TB, I), lambda g: (jnp.maximum(g - nh, 0), 0)),
            pl.BlockSpec((_HC, I), lambda g: (jnp.minimum(g, nh - 1), 0)),
            pl.BlockSpec((_HC, _BC), lambda g: (jnp.minimum(g, nh - 1), 0)),
            pl.BlockSpec((O, _HC), lambda g: (0, jnp.minimum(g, nh - 1))),
            pl.BlockSpec((1, O), lambda g: (0, 0)),
        ],
        out_specs=pl.BlockSpec((TB, O), lambda g: (jnp.maximum(g - nh, 0), 0)),
        scratch_shapes=[
            pltpu.VMEM((O, I), jnp.float32),
            pltpu.VMEM((O, _BC), jnp.float32),
            pltpu.VMEM((O, I), jnp.bfloat16),
            pltpu.VMEM((1, O), jnp.float32),
        ],
        compiler_params=pltpu.CompilerParams(
            dimension_semantics=("arbitrary",),
        ),
    )(x, w1, b1c, w2, b2.reshape(1, O).astype(jnp.float32))
    return out


# speedup vs baseline: 1.2445x; 1.1723x over previous
import functools

import jax
import jax.numpy as jnp
from jax.experimental import pallas as pl
from jax.experimental.pallas import tpu as pltpu

_HC = 1024


def _fused_body(nh, x_ref, w1_ref, b1r_ref, w2_ref, b2_ref, o_ref,
                mct_ref, bct_ref, mcb_ref, bias_ref):
    g = pl.program_id(0)

    @pl.when(g < nh)
    def _compose():
        w2c = w2_ref[...].astype(jnp.bfloat16)
        w1c = w1_ref[...].astype(jnp.bfloat16)
        part = jax.lax.dot_general(
            w2c, w1c, dimension_numbers=(((1,), (0,)), ((), ())),
            preferred_element_type=jnp.float32,
        )
        partb = jax.lax.dot_general(
            b1r_ref[...].astype(jnp.bfloat16), w2c,
            dimension_numbers=(((1,), (1,)), ((), ())),
            preferred_element_type=jnp.float32,
        )

        @pl.when(g == 0)
        def _():
            mct_ref[...] = part
            bct_ref[...] = partb

        @pl.when(g > 0)
        def _():
            mct_ref[...] += part
            bct_ref[...] += partb

    @pl.when(g >= nh)
    def _apply():
        @pl.when(g == nh)
        def _():
            mcb_ref[...] = mct_ref[...].astype(jnp.bfloat16)
            bias_ref[...] = bct_ref[...] + b2_ref[...]

        xb = x_ref[...].astype(jnp.bfloat16)
        acc = jax.lax.dot_general(
            xb, mcb_ref[...], dimension_numbers=(((1,), (1,)), ((), ())),
            preferred_element_type=jnp.float32,
        )
        o_ref[...] = acc + bias_ref[...]


def kernel(x, w1, b1, w2, b2):
    B, I = x.shape
    H = w1.shape[0]
    O = w2.shape[0]

    nh = H // _HC
    TB = min(1024, B)
    nb = B // TB

    out = pl.pallas_call(
        functools.partial(_fused_body, nh),
        out_shape=jax.ShapeDtypeStruct((B, O), jnp.float32),
        grid=(nh + nb,),
        in_specs=[
            pl.BlockSpec((TB, I), lambda g: (jnp.maximum(g - nh, 0), 0)),
            pl.BlockSpec((_HC, I), lambda g: (jnp.minimum(g, nh - 1), 0)),
            pl.BlockSpec((1, _HC), lambda g: (0, jnp.minimum(g, nh - 1))),
            pl.BlockSpec((O, _HC), lambda g: (0, jnp.minimum(g, nh - 1))),
            pl.BlockSpec((1, O), lambda g: (0, 0)),
        ],
        out_specs=pl.BlockSpec((TB, O), lambda g: (jnp.maximum(g - nh, 0), 0)),
        scratch_shapes=[
            pltpu.VMEM((O, I), jnp.float32),
            pltpu.VMEM((1, O), jnp.float32),
            pltpu.VMEM((O, I), jnp.bfloat16),
            pltpu.VMEM((1, O), jnp.float32),
        ],
        compiler_params=pltpu.CompilerParams(
            dimension_semantics=("arbitrary",),
        ),
    )(x, w1, b1.reshape(1, H), w2, b2.reshape(1, O).astype(jnp.float32))
    return out
